# baseline (speedup 1.0000x reference)
import functools

import jax
import jax.numpy as jnp
from jax.experimental import pallas as pl
from jax.experimental.pallas import tpu as pltpu
from jax.experimental.pallas import tpu_sc as plsc

B = 256
C_IN = 32
T0 = 2048
U = 512
HIDDEN = 64
OUT = 32
KNN = 8

CB = 4
GL = 4
GB = 32


def _mm(a, w):
    return jax.lax.dot_general(a, w, (((1,), (0,)), ((), ())),
                               preferred_element_type=jnp.float32)


def _conv_body(x_ref, w1lo_ref, w1mid_ref, w1hi_ref, b1_ref,
               w2lo_ref, w2mid_ref, w2hi_ref, b2_ref, out_ref):
    X = x_ref[0]
    iota = jax.lax.broadcasted_iota(jnp.int32, (4 * U, 1), 0)
    first = (iota & (U - 1)) == 0
    last = (iota & (U - 1)) == U - 1

    def up(a):
        return jnp.where(first, 0.0, jnp.roll(a, 1, axis=0))

    def dn(a):
        return jnp.where(last, 0.0, jnp.roll(a, -1, axis=0))

    o1 = (_mm(X, w1mid_ref[...]) + _mm(up(X), w1lo_ref[...])
          + _mm(dn(X), w1hi_ref[...]) + b1_ref[0])
    a = jnp.maximum(o1, 0.0)
    p1 = jnp.maximum(a[:, 0:64], a[:, 64:128])
    o2 = (_mm(p1, w2mid_ref[...]) + _mm(up(p1), w2lo_ref[...])
          + _mm(dn(p1), w2hi_ref[...]) + b2_ref[0])
    r = jnp.maximum(o2, 0.0)
    hc = jnp.maximum(r[:, 0:64], r[:, 64:128])
    for p in range(4):
        out_ref[2 * p] = hc[U * p:U * (p + 1), 0:32]
        out_ref[2 * p + 1] = hc[U * p:U * (p + 1), 32:64]


def _graph_body(h0_ref, idx_ref):
    h0 = h0_ref[...]
    g = jax.lax.dot_general(h0, h0, (((1,), (1,)), ((), ())),
                            preferred_element_type=jnp.float32)
    sqc = jnp.sum(h0 * h0, axis=1, keepdims=True)
    sqr = jnp.sum(h0 * h0, axis=1)[None, :]
    row = jax.lax.broadcasted_iota(jnp.int32, (U, U), 0)
    col = jax.lax.broadcasted_iota(jnp.int32, (U, U), 1)
    eye = row == col
    d2 = sqc + sqr - 2.0 * g + jnp.where(eye, 1e9, 0.0)
    col16 = jax.lax.broadcasted_iota(jnp.int32, (U, 16), 1)
    row16 = jax.lax.broadcasted_iota(jnp.int32, (U, 16), 0)
    acc16 = jnp.where(col16 == KNN, row16, 0)
    for k in range(KNN):
        m = jnp.min(d2, axis=1, keepdims=True)
        cand = jnp.where(d2 == m, col, U)
        idx = jnp.min(cand, axis=1, keepdims=True)
        sel = col == idx
        acc16 = jnp.where(col16 == k, idx, acc16)
        d2 = jnp.where(sel, 3e9, d2)
    idx_ref[...] = acc16


_SC_NC = 2
_SC_NS = 16
_ROWS_PER_W = U // (_SC_NC * _SC_NS)


def _build_m_sc(idx16):
    mesh = plsc.VectorSubcoreMesh(core_axis_name="c", subcore_axis_name="s")

    @functools.partial(
        pl.kernel,
        mesh=mesh,
        out_type=jax.ShapeDtypeStruct((U, U), jnp.float32),
        scratch_types=[
            pltpu.VMEM((_ROWS_PER_W, 16), jnp.int32),
            pltpu.VMEM((U,), jnp.float32),
        ],
        compiler_params=pltpu.CompilerParams(needs_layout_passes=False),
    )
    def mbuild(idx_hbm, m_hbm, idx_v, row_v):
        c = jax.lax.axis_index("c")
        s = jax.lax.axis_index("s")
        base = (s * _SC_NC + c) * _ROWS_PER_W
        pltpu.sync_copy(idx_hbm.at[pl.ds(base, _ROWS_PER_W)], idx_v)
        lane = jax.lax.broadcasted_iota(jnp.int32, (16,), 0)
        mask = lane < KNN + 1
        vals = jnp.full((16,), 1.0 / 9.0, dtype=jnp.float32)
        zeros16 = jnp.zeros((16,), jnp.float32)
        for r in range(_ROWS_PER_W):
            for z in range(U // 16):
                row_v[pl.ds(z * 16, 16)] = zeros16
            plsc.store_scatter(row_v, [idx_v[r]], vals, mask=mask)
            pltpu.sync_copy(row_v, m_hbm.at[base + r])

    return mbuild(idx16)


def _gcn_body(m_ref, h_ref, w1_ref, b1_ref, w2_ref, b2_ref,
              fcw_ref, fcb_ref, out_ref):
    mop = m_ref[...]
    pooled = []
    for g in range(GB // GL):
        h4 = jnp.concatenate([h_ref[GL * g + i] for i in range(GL)],
                             axis=1)
        hw = _mm(h4, w1_ref[...])
        a1 = jnp.maximum(_mm(mop, hw) + b1_ref[0], 0.0)
        a2 = jnp.maximum(_mm(mop, _mm(a1, w2_ref[...])) + b2_ref[0], 0.0)
        pooled.append(jnp.sum(a2, axis=0, keepdims=True) * (1.0 / U))
    pooled = jnp.concatenate(pooled, axis=0)
    out_ref[...] = _mm(pooled, fcw_ref[...]) + fcb_ref[0]


def kernel(x, conv1_w, conv1_b, conv2_w, conv2_b,
           gcn1_w, gcn1_b, gcn2_w, gcn2_b, fc_w, fc_b):
    f32 = jnp.float32
    xbig = x.reshape(B // 8, 4, 2, C_IN, U, 4) \
            .transpose(0, 1, 4, 2, 5, 3).reshape(B // 8, 4 * U, 256)

    w1mid = jnp.zeros((256, 128), f32)
    w1lo = jnp.zeros((256, 128), f32)
    w1hi = jnp.zeros((256, 128), f32)

    def _cs1(h, j):
        return 16 * ((j // 2 if j % 2 == 0 else 4 + j // 2) + 2 * h)

    for h in range(2):
        for j in range(4):
            col = _cs1(h, j)
            for dt in range(5):
                blk = j + dt
                wT = conv1_w[:, :, dt].T
                if 2 <= blk <= 5:
                    w1mid = w1mid.at[128 * h + 32 * (blk - 2):
                                     128 * h + 32 * (blk - 1),
                                     col:col + 16].set(wT)
                elif blk < 2:
                    w1lo = w1lo.at[128 * h + 64 + 32 * blk:
                                   128 * h + 96 + 32 * blk,
                                   col:col + 16].set(wT)
                else:
                    w1hi = w1hi.at[128 * h + 32 * (blk - 6):
                                   128 * h + 32 * (blk - 5),
                                   col:col + 16].set(wT)
    w2mid = jnp.zeros((64, 128), f32)
    w2lo = jnp.zeros((64, 128), f32)
    w2hi = jnp.zeros((64, 128), f32)
    for h in range(2):
        for j in range(2):
            col = 64 * j + 32 * h
            for dt in range(5):
                blk = j + dt
                wT = conv2_w[:, :, dt].T
                if 2 <= blk <= 3:
                    w2mid = w2mid.at[32 * h + 16 * (blk - 2):
                                     32 * h + 16 * (blk - 1),
                                     col:col + 32].set(wT)
                elif blk < 2:
                    w2lo = w2lo.at[32 * h + 16 * blk:
                                   32 * h + 16 * (blk + 1),
                                   col:col + 32].set(wT)
                else:
                    w2hi = w2hi.at[32 * h + 16 * (blk - 4):
                                   32 * h + 16 * (blk - 3),
                                   col:col + 32].set(wT)
    b1c = jnp.tile(conv1_b, 8)[None, :]
    b2c = jnp.tile(conv2_b, 4)[None, :]

    w1blk = jnp.zeros((GL * C_IN, GL * HIDDEN), f32)
    w2blk = jnp.zeros((GL * HIDDEN, GL * HIDDEN), f32)
    fcblk = jnp.zeros((GL * HIDDEN, GL * OUT), f32)
    for g in range(GL):
        w1blk = w1blk.at[C_IN * g:C_IN * (g + 1),
                         HIDDEN * g:HIDDEN * (g + 1)].set(gcn1_w)
        w2blk = w2blk.at[HIDDEN * g:HIDDEN * (g + 1),
                         HIDDEN * g:HIDDEN * (g + 1)].set(gcn2_w)
        fcblk = fcblk.at[HIDDEN * g:HIDDEN * (g + 1),
                         OUT * g:OUT * (g + 1)].set(fc_w.T)
    g1bt = jnp.tile(gcn1_b, GL)[None, :]
    g2bt = jnp.tile(gcn2_b, GL)[None, :]
    fcbt = jnp.tile(fc_b, GL)[None, :]

    h = pl.pallas_call(
        _conv_body,
        grid=(B // 8,),
        in_specs=[
            pl.BlockSpec((1, 4 * U, 256), lambda b: (b, 0, 0)),
            pl.BlockSpec((256, 128), lambda b: (0, 0)),
            pl.BlockSpec((256, 128), lambda b: (0, 0)),
            pl.BlockSpec((256, 128), lambda b: (0, 0)),
            pl.BlockSpec((1, 128), lambda b: (0, 0)),
            pl.BlockSpec((64, 128), lambda b: (0, 0)),
            pl.BlockSpec((64, 128), lambda b: (0, 0)),
            pl.BlockSpec((64, 128), lambda b: (0, 0)),
            pl.BlockSpec((1, 128), lambda b: (0, 0)),
        ],
        out_specs=pl.BlockSpec((8, U, C_IN), lambda b: (b, 0, 0)),
        out_shape=jax.ShapeDtypeStruct((B, U, C_IN), jnp.float32),
    )(xbig, w1lo, w1mid, w1hi, b1c, w2lo, w2mid, w2hi, b2c)

    idx16 = pl.pallas_call(
        _graph_body,
        out_shape=jax.ShapeDtypeStruct((U, 16), jnp.int32),
    )(h[0])
    mop = _build_m_sc(idx16)

    out = pl.pallas_call(
        _gcn_body,
        grid=(B // GB,),
        in_specs=[
            pl.BlockSpec((U, U), lambda b: (0, 0)),
            pl.BlockSpec((GB, U, C_IN), lambda b: (b, 0, 0)),
            pl.BlockSpec((GL * C_IN, GL * HIDDEN), lambda b: (0, 0)),
            pl.BlockSpec((1, GL * HIDDEN), lambda b: (0, 0)),
            pl.BlockSpec((GL * HIDDEN, GL * HIDDEN), lambda b: (0, 0)),
            pl.BlockSpec((1, GL * HIDDEN), lambda b: (0, 0)),
            pl.BlockSpec((GL * HIDDEN, GL * OUT), lambda b: (0, 0)),
            pl.BlockSpec((1, GL * OUT), lambda b: (0, 0)),
        ],
        out_specs=pl.BlockSpec((GB // GL, GL * OUT), lambda b: (b, 0)),
        out_shape=jax.ShapeDtypeStruct((B // GL, GL * OUT), jnp.float32),
    )(mop, h, w1blk, g1bt, w2blk, g2bt, fcblk, fcbt)
    return out.reshape(B, OUT)


# speedup vs baseline: 1.5006x; 1.5006x over previous
import functools

import jax
import jax.numpy as jnp
from jax.experimental import pallas as pl
from jax.experimental.pallas import tpu as pltpu
from jax.experimental.pallas import tpu_sc as plsc

B = 256
C_IN = 32
T0 = 2048
U = 512
HIDDEN = 64
OUT = 32
KNN = 8

CB = 4
GL = 4
GB = 32


def _mm(a, w):
    return jax.lax.dot_general(a, w, (((1,), (0,)), ((), ())),
                               preferred_element_type=jnp.float32)


def _conv_body(x_ref, w1lo_ref, w1mid_ref, w1hi_ref, b1_ref,
               w2lo_ref, w2mid_ref, w2hi_ref, b2_ref, out_ref):
    X = jnp.concatenate(
        [x_ref[0:4].reshape(4 * U, 128),
         x_ref[4:8].reshape(4 * U, 128)], axis=1)
    iota = jax.lax.broadcasted_iota(jnp.int32, (4 * U, 1), 0)
    first = (iota & (U - 1)) == 0
    last = (iota & (U - 1)) == U - 1

    def up(a):
        return jnp.where(first, 0.0, jnp.roll(a, 1, axis=0))

    def dn(a):
        return jnp.where(last, 0.0, jnp.roll(a, -1, axis=0))

    o1 = (_mm(X, w1mid_ref[...]) + _mm(up(X), w1lo_ref[...])
          + _mm(dn(X), w1hi_ref[...]) + b1_ref[0])
    a = jnp.maximum(o1, 0.0)
    p1 = jnp.maximum(a[:, 0:64], a[:, 64:128])
    o2 = (_mm(p1, w2mid_ref[...]) + _mm(up(p1), w2lo_ref[...])
          + _mm(dn(p1), w2hi_ref[...]) + b2_ref[0])
    r = jnp.maximum(o2, 0.0)
    hc = jnp.maximum(r[:, 0:64], r[:, 64:128])
    for p in range(4):
        out_ref[p] = hc[U * p:U * (p + 1), 0:32]
        out_ref[p + 4] = hc[U * p:U * (p + 1), 32:64]


def _graph_body(h0_ref, idx_ref):
    h0 = h0_ref[...]
    g = jax.lax.dot_general(h0, h0, (((1,), (1,)), ((), ())),
                            preferred_element_type=jnp.float32)
    sqc = jnp.sum(h0 * h0, axis=1, keepdims=True)
    sqr = jnp.sum(h0 * h0, axis=1)[None, :]
    row = jax.lax.broadcasted_iota(jnp.int32, (U, U), 0)
    col = jax.lax.broadcasted_iota(jnp.int32, (U, U), 1)
    eye = row == col
    d2 = sqc + sqr - 2.0 * g + jnp.where(eye, 1e9, 0.0)
    col16 = jax.lax.broadcasted_iota(jnp.int32, (U, 16), 1)
    row16 = jax.lax.broadcasted_iota(jnp.int32, (U, 16), 0)
    acc16 = jnp.where(col16 == KNN, row16, 0)
    for k in range(KNN):
        m = jnp.min(d2, axis=1, keepdims=True)
        cand = jnp.where(d2 == m, col, U)
        idx = jnp.min(cand, axis=1, keepdims=True)
        sel = col == idx
        acc16 = jnp.where(col16 == k, idx, acc16)
        d2 = jnp.where(sel, 3e9, d2)
    idx_ref[...] = acc16


_SC_NC = 2
_SC_NS = 16
_ROWS_PER_W = U // (_SC_NC * _SC_NS)


def _build_m_sc(idx16):
    mesh = plsc.VectorSubcoreMesh(core_axis_name="c", subcore_axis_name="s")

    @functools.partial(
        pl.kernel,
        mesh=mesh,
        out_type=jax.ShapeDtypeStruct((U, U), jnp.float32),
        scratch_types=[
            pltpu.VMEM((_ROWS_PER_W, 16), jnp.int32),
            pltpu.VMEM((U,), jnp.float32),
        ],
        compiler_params=pltpu.CompilerParams(needs_layout_passes=False),
    )
    def mbuild(idx_hbm, m_hbm, idx_v, row_v):
        c = jax.lax.axis_index("c")
        s = jax.lax.axis_index("s")
        base = (s * _SC_NC + c) * _ROWS_PER_W
        pltpu.sync_copy(idx_hbm.at[pl.ds(base, _ROWS_PER_W)], idx_v)
        lane = jax.lax.broadcasted_iota(jnp.int32, (16,), 0)
        mask = lane < KNN + 1
        vals = jnp.full((16,), 1.0 / 9.0, dtype=jnp.float32)
        zeros16 = jnp.zeros((16,), jnp.float32)
        for r in range(_ROWS_PER_W):
            for z in range(U // 16):
                row_v[pl.ds(z * 16, 16)] = zeros16
            plsc.store_scatter(row_v, [idx_v[r]], vals, mask=mask)
            pltpu.sync_copy(row_v, m_hbm.at[base + r])

    return mbuild(idx16)


def _gcn_body(m_ref, h_ref, w1_ref, b1_ref, w2_ref, b2_ref,
              fcw_ref, fcb_ref, out_ref):
    mop = m_ref[...]
    pooled = []
    for g in range(GB // GL):
        h4 = jnp.concatenate([h_ref[GL * g + i] for i in range(GL)],
                             axis=1)
        hw = _mm(h4, w1_ref[...])
        a1 = jnp.maximum(_mm(mop, hw) + b1_ref[0], 0.0)
        a2 = jnp.maximum(_mm(mop, _mm(a1, w2_ref[...])) + b2_ref[0], 0.0)
        pooled.append(jnp.sum(a2, axis=0, keepdims=True) * (1.0 / U))
    pooled = jnp.concatenate(pooled, axis=0)
    out_ref[...] = _mm(pooled, fcw_ref[...]) + fcb_ref[0]


def kernel(x, conv1_w, conv1_b, conv2_w, conv2_b,
           gcn1_w, gcn1_b, gcn2_w, gcn2_b, fc_w, fc_b):
    f32 = jnp.float32
    xcat = jnp.transpose(x.reshape(B, C_IN, U, 4), (0, 2, 3, 1)) \
              .reshape(B, U, 4 * C_IN)

    w1mid = jnp.zeros((256, 128), f32)
    w1lo = jnp.zeros((256, 128), f32)
    w1hi = jnp.zeros((256, 128), f32)

    def _cs1(h, j):
        return 16 * ((j // 2 if j % 2 == 0 else 4 + j // 2) + 2 * h)

    for h in range(2):
        for j in range(4):
            col = _cs1(h, j)
            for dt in range(5):
                blk = j + dt
                wT = conv1_w[:, :, dt].T
                if 2 <= blk <= 5:
                    w1mid = w1mid.at[128 * h + 32 * (blk - 2):
                                     128 * h + 32 * (blk - 1),
                                     col:col + 16].set(wT)
                elif blk < 2:
                    w1lo = w1lo.at[128 * h + 64 + 32 * blk:
                                   128 * h + 96 + 32 * blk,
                                   col:col + 16].set(wT)
                else:
                    w1hi = w1hi.at[128 * h + 32 * (blk - 6):
                                   128 * h + 32 * (blk - 5),
                                   col:col + 16].set(wT)
    w2mid = jnp.zeros((64, 128), f32)
    w2lo = jnp.zeros((64, 128), f32)
    w2hi = jnp.zeros((64, 128), f32)
    for h in range(2):
        for j in range(2):
            col = 64 * j + 32 * h
            for dt in range(5):
                blk = j + dt
                wT = conv2_w[:, :, dt].T
                if 2 <= blk <= 3:
                    w2mid = w2mid.at[32 * h + 16 * (blk - 2):
                                     32 * h + 16 * (blk - 1),
                                     col:col + 32].set(wT)
                elif blk < 2:
                    w2lo = w2lo.at[32 * h + 16 * blk:
                                   32 * h + 16 * (blk + 1),
                                   col:col + 32].set(wT)
                else:
                    w2hi = w2hi.at[32 * h + 16 * (blk - 4):
                                   32 * h + 16 * (blk - 3),
                                   col:col + 32].set(wT)
    b1c = jnp.tile(conv1_b, 8)[None, :]
    b2c = jnp.tile(conv2_b, 4)[None, :]

    w1blk = jnp.zeros((GL * C_IN, GL * HIDDEN), f32)
    w2blk = jnp.zeros((GL * HIDDEN, GL * HIDDEN), f32)
    fcblk = jnp.zeros((GL * HIDDEN, GL * OUT), f32)
    for g in range(GL):
        w1blk = w1blk.at[C_IN * g:C_IN * (g + 1),
                         HIDDEN * g:HIDDEN * (g + 1)].set(gcn1_w)
        w2blk = w2blk.at[HIDDEN * g:HIDDEN * (g + 1),
                         HIDDEN * g:HIDDEN * (g + 1)].set(gcn2_w)
        fcblk = fcblk.at[HIDDEN * g:HIDDEN * (g + 1),
                         OUT * g:OUT * (g + 1)].set(fc_w.T)
    g1bt = jnp.tile(gcn1_b, GL)[None, :]
    g2bt = jnp.tile(gcn2_b, GL)[None, :]
    fcbt = jnp.tile(fc_b, GL)[None, :]

    h = pl.pallas_call(
        _conv_body,
        grid=(B // 8,),
        in_specs=[
            pl.BlockSpec((8, U, 4 * C_IN), lambda b: (b, 0, 0)),
            pl.BlockSpec((256, 128), lambda b: (0, 0)),
            pl.BlockSpec((256, 128), lambda b: (0, 0)),
            pl.BlockSpec((256, 128), lambda b: (0, 0)),
            pl.BlockSpec((1, 128), lambda b: (0, 0)),
            pl.BlockSpec((64, 128), lambda b: (0, 0)),
            pl.BlockSpec((64, 128), lambda b: (0, 0)),
            pl.BlockSpec((64, 128), lambda b: (0, 0)),
            pl.BlockSpec((1, 128), lambda b: (0, 0)),
        ],
        out_specs=pl.BlockSpec((8, U, C_IN), lambda b: (b, 0, 0)),
        out_shape=jax.ShapeDtypeStruct((B, U, C_IN), jnp.float32),
    )(xcat, w1lo, w1mid, w1hi, b1c, w2lo, w2mid, w2hi, b2c)

    idx16 = pl.pallas_call(
        _graph_body,
        out_shape=jax.ShapeDtypeStruct((U, 16), jnp.int32),
    )(h[0])
    mop = _build_m_sc(idx16)

    out = pl.pallas_call(
        _gcn_body,
        grid=(B // GB,),
        in_specs=[
            pl.BlockSpec((U, U), lambda b: (0, 0)),
            pl.BlockSpec((GB, U, C_IN), lambda b: (b, 0, 0)),
            pl.BlockSpec((GL * C_IN, GL * HIDDEN), lambda b: (0, 0)),
            pl.BlockSpec((1, GL * HIDDEN), lambda b: (0, 0)),
            pl.BlockSpec((GL * HIDDEN, GL * HIDDEN), lambda b: (0, 0)),
            pl.BlockSpec((1, GL * HIDDEN), lambda b: (0, 0)),
            pl.BlockSpec((GL * HIDDEN, GL * OUT), lambda b: (0, 0)),
            pl.BlockSpec((1, GL * OUT), lambda b: (0, 0)),
        ],
        out_specs=pl.BlockSpec((GB // GL, GL * OUT), lambda b: (b, 0)),
        out_shape=jax.ShapeDtypeStruct((B // GL, GL * OUT), jnp.float32),
    )(mop, h, w1blk, g1bt, w2blk, g2bt, fcblk, fcbt)
    return out.reshape(B, OUT)


# speedup vs baseline: 1.5622x; 1.0411x over previous
import functools

import jax
import jax.numpy as jnp
from jax.experimental import pallas as pl
from jax.experimental.pallas import tpu as pltpu
from jax.experimental.pallas import tpu_sc as plsc

B = 256
C_IN = 32
T0 = 2048
U = 512
HIDDEN = 64
OUT = 32
KNN = 8

CB = 4
GL = 4
GB = 32


def _mm(a, w):
    return jax.lax.dot_general(a, w, (((1,), (0,)), ((), ())),
                               preferred_element_type=jnp.float32)


CS = 16


def _conv_body(x_ref, w1lo_ref, w1mid_ref, w1hi_ref, b1_ref,
               w2lo_ref, w2mid_ref, w2hi_ref, b2_ref, out_ref, idx_ref):
    NP = CS // 2
    X = jnp.concatenate(
        [x_ref[0:NP].reshape(NP * U, 128),
         x_ref[NP:CS].reshape(NP * U, 128)], axis=1)
    iota = jax.lax.broadcasted_iota(jnp.int32, (NP * U, 1), 0)
    first = (iota & (U - 1)) == 0
    last = (iota & (U - 1)) == U - 1

    def up(a):
        return jnp.where(first, 0.0, jnp.roll(a, 1, axis=0))

    def dn(a):
        return jnp.where(last, 0.0, jnp.roll(a, -1, axis=0))

    o1 = (_mm(X, w1mid_ref[...]) + _mm(up(X), w1lo_ref[...])
          + _mm(dn(X), w1hi_ref[...]) + b1_ref[0])
    a = jnp.maximum(o1, 0.0)
    p1 = jnp.maximum(a[:, 0:64], a[:, 64:128])
    o2 = (_mm(p1, w2mid_ref[...]) + _mm(up(p1), w2lo_ref[...])
          + _mm(dn(p1), w2hi_ref[...]) + b2_ref[0])
    r = jnp.maximum(o2, 0.0)
    hc = jnp.maximum(r[:, 0:64], r[:, 64:128])
    for p in range(NP):
        out_ref[p] = hc[U * p:U * (p + 1), 0:32]
        out_ref[p + NP] = hc[U * p:U * (p + 1), 32:64]

    @pl.when(pl.program_id(0) == 0)
    def _():
        _graph_from_h0(hc[0:U, 0:32], idx_ref)


def _graph_from_h0(h0, idx_ref):
    g = jax.lax.dot_general(h0, h0, (((1,), (1,)), ((), ())),
                            preferred_element_type=jnp.float32)
    sqc = jnp.sum(h0 * h0, axis=1, keepdims=True)
    sqr = jnp.sum(h0 * h0, axis=1)[None, :]
    row = jax.lax.broadcasted_iota(jnp.int32, (U, U), 0)
    col = jax.lax.broadcasted_iota(jnp.int32, (U, U), 1)
    eye = row == col
    d2 = sqc + sqr - 2.0 * g + jnp.where(eye, 1e9, 0.0)
    col16 = jax.lax.broadcasted_iota(jnp.int32, (U, 16), 1)
    row16 = jax.lax.broadcasted_iota(jnp.int32, (U, 16), 0)
    acc16 = jnp.where(col16 == KNN, row16, 0)
    for k in range(KNN):
        m = jnp.min(d2, axis=1, keepdims=True)
        cand = jnp.where(d2 == m, col, U)
        idx = jnp.min(cand, axis=1, keepdims=True)
        sel = col == idx
        acc16 = jnp.where(col16 == k, idx, acc16)
        d2 = jnp.where(sel, 3e9, d2)
    idx_ref[...] = acc16


_SC_NC = 2
_SC_NS = 16
_ROWS_PER_W = U // (_SC_NC * _SC_NS)


def _build_m_sc(idx16):
    mesh = plsc.VectorSubcoreMesh(core_axis_name="c", subcore_axis_name="s")

    @functools.partial(
        pl.kernel,
        mesh=mesh,
        out_type=jax.ShapeDtypeStruct((U, U), jnp.float32),
        scratch_types=[
            pltpu.VMEM((_ROWS_PER_W, 16), jnp.int32),
            pltpu.VMEM((U,), jnp.float32),
        ],
        compiler_params=pltpu.CompilerParams(needs_layout_passes=False),
    )
    def mbuild(idx_hbm, m_hbm, idx_v, row_v):
        c = jax.lax.axis_index("c")
        s = jax.lax.axis_index("s")
        base = (s * _SC_NC + c) * _ROWS_PER_W
        pltpu.sync_copy(idx_hbm.at[pl.ds(base, _ROWS_PER_W)], idx_v)
        lane = jax.lax.broadcasted_iota(jnp.int32, (16,), 0)
        mask = lane < KNN + 1
        vals = jnp.full((16,), 1.0 / 9.0, dtype=jnp.float32)
        zeros16 = jnp.zeros((16,), jnp.float32)
        for r in range(_ROWS_PER_W):
            for z in range(U // 16):
                row_v[pl.ds(z * 16, 16)] = zeros16
            plsc.store_scatter(row_v, [idx_v[r]], vals, mask=mask)
            pltpu.sync_copy(row_v, m_hbm.at[base + r])

    return mbuild(idx16)


def _gcn_body(m_ref, h_ref, w1_ref, b1_ref, w2_ref, b2_ref,
              fcw_ref, fcb_ref, out_ref):
    mop = m_ref[...]
    pooled = []
    for g in range(GB // GL):
        h4 = jnp.concatenate([h_ref[GL * g + i] for i in range(GL)],
                             axis=1)
        hw = _mm(h4, w1_ref[...])
        a1 = jnp.maximum(_mm(mop, hw) + b1_ref[0], 0.0)
        a2 = jnp.maximum(_mm(mop, _mm(a1, w2_ref[...])) + b2_ref[0], 0.0)
        pooled.append(jnp.sum(a2, axis=0, keepdims=True) * (1.0 / U))
    pooled = jnp.concatenate(pooled, axis=0)
    out_ref[...] = _mm(pooled, fcw_ref[...]) + fcb_ref[0]


def kernel(x, conv1_w, conv1_b, conv2_w, conv2_b,
           gcn1_w, gcn1_b, gcn2_w, gcn2_b, fc_w, fc_b):
    f32 = jnp.float32
    xcat = jnp.transpose(x.reshape(B, C_IN, U, 4), (0, 2, 3, 1)) \
              .reshape(B, U, 4 * C_IN)

    w1mid = jnp.zeros((256, 128), f32)
    w1lo = jnp.zeros((256, 128), f32)
    w1hi = jnp.zeros((256, 128), f32)

    def _cs1(h, j):
        return 16 * ((j // 2 if j % 2 == 0 else 4 + j // 2) + 2 * h)

    for h in range(2):
        for j in range(4):
            col = _cs1(h, j)
            for dt in range(5):
                blk = j + dt
                wT = conv1_w[:, :, dt].T
                if 2 <= blk <= 5:
                    w1mid = w1mid.at[128 * h + 32 * (blk - 2):
                                     128 * h + 32 * (blk - 1),
                                     col:col + 16].set(wT)
                elif blk < 2:
                    w1lo = w1lo.at[128 * h + 64 + 32 * blk:
                                   128 * h + 96 + 32 * blk,
                                   col:col + 16].set(wT)
                else:
                    w1hi = w1hi.at[128 * h + 32 * (blk - 6):
                                   128 * h + 32 * (blk - 5),
                                   col:col + 16].set(wT)
    w2mid = jnp.zeros((64, 128), f32)
    w2lo = jnp.zeros((64, 128), f32)
    w2hi = jnp.zeros((64, 128), f32)
    for h in range(2):
        for j in range(2):
            col = 64 * j + 32 * h
            for dt in range(5):
                blk = j + dt
                wT = conv2_w[:, :, dt].T
                if 2 <= blk <= 3:
                    w2mid = w2mid.at[32 * h + 16 * (blk - 2):
                                     32 * h + 16 * (blk - 1),
                                     col:col + 32].set(wT)
                elif blk < 2:
                    w2lo = w2lo.at[32 * h + 16 * blk:
                                   32 * h + 16 * (blk + 1),
                                   col:col + 32].set(wT)
                else:
                    w2hi = w2hi.at[32 * h + 16 * (blk - 4):
                                   32 * h + 16 * (blk - 3),
                                   col:col + 32].set(wT)
    b1c = jnp.tile(conv1_b, 8)[None, :]
    b2c = jnp.tile(conv2_b, 4)[None, :]

    w1blk = jnp.zeros((GL * C_IN, GL * HIDDEN), f32)
    w2blk = jnp.zeros((GL * HIDDEN, GL * HIDDEN), f32)
    fcblk = jnp.zeros((GL * HIDDEN, GL * OUT), f32)
    for g in range(GL):
        w1blk = w1blk.at[C_IN * g:C_IN * (g + 1),
                         HIDDEN * g:HIDDEN * (g + 1)].set(gcn1_w)
        w2blk = w2blk.at[HIDDEN * g:HIDDEN * (g + 1),
                         HIDDEN * g:HIDDEN * (g + 1)].set(gcn2_w)
        fcblk = fcblk.at[HIDDEN * g:HIDDEN * (g + 1),
                         OUT * g:OUT * (g + 1)].set(fc_w.T)
    g1bt = jnp.tile(gcn1_b, GL)[None, :]
    g2bt = jnp.tile(gcn2_b, GL)[None, :]
    fcbt = jnp.tile(fc_b, GL)[None, :]

    h, idx16 = pl.pallas_call(
        _conv_body,
        grid=(B // CS,),
        in_specs=[
            pl.BlockSpec((CS, U, 4 * C_IN), lambda b: (b, 0, 0)),
            pl.BlockSpec((256, 128), lambda b: (0, 0)),
            pl.BlockSpec((256, 128), lambda b: (0, 0)),
            pl.BlockSpec((256, 128), lambda b: (0, 0)),
            pl.BlockSpec((1, 128), lambda b: (0, 0)),
            pl.BlockSpec((64, 128), lambda b: (0, 0)),
            pl.BlockSpec((64, 128), lambda b: (0, 0)),
            pl.BlockSpec((64, 128), lambda b: (0, 0)),
            pl.BlockSpec((1, 128), lambda b: (0, 0)),
        ],
        out_specs=[
            pl.BlockSpec((CS, U, C_IN), lambda b: (b, 0, 0)),
            pl.BlockSpec((U, 16), lambda b: (0, 0)),
        ],
        out_shape=[
            jax.ShapeDtypeStruct((B, U, C_IN), jnp.float32),
            jax.ShapeDtypeStruct((U, 16), jnp.int32),
        ],
    )(xcat, w1lo, w1mid, w1hi, b1c, w2lo, w2mid, w2hi, b2c)

    mop = _build_m_sc(idx16)

    out = pl.pallas_call(
        _gcn_body,
        grid=(B // GB,),
        in_specs=[
            pl.BlockSpec((U, U), lambda b: (0, 0)),
            pl.BlockSpec((GB, U, C_IN), lambda b: (b, 0, 0)),
            pl.BlockSpec((GL * C_IN, GL * HIDDEN), lambda b: (0, 0)),
            pl.BlockSpec((1, GL * HIDDEN), lambda b: (0, 0)),
            pl.BlockSpec((GL * HIDDEN, GL * HIDDEN), lambda b: (0, 0)),
            pl.BlockSpec((1, GL * HIDDEN), lambda b: (0, 0)),
            pl.BlockSpec((GL * HIDDEN, GL * OUT), lambda b: (0, 0)),
            pl.BlockSpec((1, GL * OUT), lambda b: (0, 0)),
        ],
        out_specs=pl.BlockSpec((GB // GL, GL * OUT), lambda b: (b, 0)),
        out_shape=jax.ShapeDtypeStruct((B // GL, GL * OUT), jnp.float32),
    )(mop, h, w1blk, g1bt, w2blk, g2bt, fcblk, fcbt)
    return out.reshape(B, OUT)


# speedup vs baseline: 1.5626x; 1.0003x over previous
import functools

import jax
import jax.numpy as jnp
from jax.experimental import pallas as pl
from jax.experimental.pallas import tpu as pltpu
from jax.experimental.pallas import tpu_sc as plsc

B = 256
C_IN = 32
T0 = 2048
U = 512
HIDDEN = 64
OUT = 32
KNN = 8

CB = 4
GL = 4
GB = 64


def _mm(a, w):
    return jax.lax.dot_general(a, w, (((1,), (0,)), ((), ())),
                               preferred_element_type=jnp.float32)


CS = 16


def _conv_body(x_ref, w1lo_ref, w1mid_ref, w1hi_ref, b1_ref,
               w2lo_ref, w2mid_ref, w2hi_ref, b2_ref, out_ref, idx_ref):
    NP = CS // 2
    X = jnp.concatenate(
        [x_ref[0:NP].reshape(NP * U, 128),
         x_ref[NP:CS].reshape(NP * U, 128)], axis=1)
    iota = jax.lax.broadcasted_iota(jnp.int32, (NP * U, 1), 0)
    first = (iota & (U - 1)) == 0
    last = (iota & (U - 1)) == U - 1

    def up(a):
        return jnp.where(first, 0.0, jnp.roll(a, 1, axis=0))

    def dn(a):
        return jnp.where(last, 0.0, jnp.roll(a, -1, axis=0))

    o1 = (_mm(X, w1mid_ref[...]) + _mm(up(X), w1lo_ref[...])
          + _mm(dn(X), w1hi_ref[...]) + b1_ref[0])
    a = jnp.maximum(o1, 0.0)
    p1 = jnp.maximum(a[:, 0:64], a[:, 64:128])
    o2 = (_mm(p1, w2mid_ref[...]) + _mm(up(p1), w2lo_ref[...])
          + _mm(dn(p1), w2hi_ref[...]) + b2_ref[0])
    r = jnp.maximum(o2, 0.0)
    hc = jnp.maximum(r[:, 0:64], r[:, 64:128])
    for p in range(NP):
        out_ref[p] = hc[U * p:U * (p + 1), 0:32]
        out_ref[p + NP] = hc[U * p:U * (p + 1), 32:64]

    @pl.when(pl.program_id(0) == 0)
    def _():
        _graph_from_h0(hc[0:U, 0:32], idx_ref)


def _graph_from_h0(h0, idx_ref):
    g = jax.lax.dot_general(h0, h0, (((1,), (1,)), ((), ())),
                            preferred_element_type=jnp.float32)
    sqc = jnp.sum(h0 * h0, axis=1, keepdims=True)
    sqr = jnp.sum(h0 * h0, axis=1)[None, :]
    row = jax.lax.broadcasted_iota(jnp.int32, (U, U), 0)
    col = jax.lax.broadcasted_iota(jnp.int32, (U, U), 1)
    eye = row == col
    d2 = sqc + sqr - 2.0 * g + jnp.where(eye, 1e9, 0.0)
    col16 = jax.lax.broadcasted_iota(jnp.int32, (U, 16), 1)
    row16 = jax.lax.broadcasted_iota(jnp.int32, (U, 16), 0)
    acc16 = jnp.where(col16 == KNN, row16, 0)
    for k in range(KNN):
        m = jnp.min(d2, axis=1, keepdims=True)
        cand = jnp.where(d2 == m, col, U)
        idx = jnp.min(cand, axis=1, keepdims=True)
        sel = col == idx
        acc16 = jnp.where(col16 == k, idx, acc16)
        d2 = jnp.where(sel, 3e9, d2)
    idx_ref[...] = acc16


_SC_NC = 2
_SC_NS = 16
_ROWS_PER_W = U // (_SC_NC * _SC_NS)


def _build_m_sc(idx16):
    mesh = plsc.VectorSubcoreMesh(core_axis_name="c", subcore_axis_name="s")

    @functools.partial(
        pl.kernel,
        mesh=mesh,
        out_type=jax.ShapeDtypeStruct((U, U), jnp.float32),
        scratch_types=[
            pltpu.VMEM((_ROWS_PER_W, 16), jnp.int32),
            pltpu.VMEM((U,), jnp.float32),
        ],
        compiler_params=pltpu.CompilerParams(needs_layout_passes=False),
    )
    def mbuild(idx_hbm, m_hbm, idx_v, row_v):
        c = jax.lax.axis_index("c")
        s = jax.lax.axis_index("s")
        base = (s * _SC_NC + c) * _ROWS_PER_W
        pltpu.sync_copy(idx_hbm.at[pl.ds(base, _ROWS_PER_W)], idx_v)
        lane = jax.lax.broadcasted_iota(jnp.int32, (16,), 0)
        mask = lane < KNN + 1
        vals = jnp.full((16,), 1.0 / 9.0, dtype=jnp.float32)
        zeros16 = jnp.zeros((16,), jnp.float32)
        for r in range(_ROWS_PER_W):
            for z in range(U // 16):
                row_v[pl.ds(z * 16, 16)] = zeros16
            plsc.store_scatter(row_v, [idx_v[r]], vals, mask=mask)
            pltpu.sync_copy(row_v, m_hbm.at[base + r])

    return mbuild(idx16)


def _gcn_body(m_ref, h_ref, w1_ref, b1_ref, w2_ref, b2_ref,
              fcw_ref, fcb_ref, out_ref):
    mop = m_ref[...]
    pooled = []
    for g in range(GB // GL):
        h4 = jnp.concatenate([h_ref[GL * g + i] for i in range(GL)],
                             axis=1)
        hw = _mm(h4, w1_ref[...])
        a1 = jnp.maximum(_mm(mop, hw) + b1_ref[0], 0.0)
        a2 = jnp.maximum(_mm(mop, _mm(a1, w2_ref[...])) + b2_ref[0], 0.0)
        pooled.append(jnp.sum(a2, axis=0, keepdims=True) * (1.0 / U))
    pooled = jnp.concatenate(pooled, axis=0)
    out_ref[...] = _mm(pooled, fcw_ref[...]) + fcb_ref[0]


def kernel(x, conv1_w, conv1_b, conv2_w, conv2_b,
           gcn1_w, gcn1_b, gcn2_w, gcn2_b, fc_w, fc_b):
    f32 = jnp.float32
    xcat = jnp.transpose(x.reshape(B, C_IN, U, 4), (0, 2, 3, 1)) \
              .reshape(B, U, 4 * C_IN)

    w1mid = jnp.zeros((256, 128), f32)
    w1lo = jnp.zeros((256, 128), f32)
    w1hi = jnp.zeros((256, 128), f32)

    def _cs1(h, j):
        return 16 * ((j // 2 if j % 2 == 0 else 4 + j // 2) + 2 * h)

    for h in range(2):
        for j in range(4):
            col = _cs1(h, j)
            for dt in range(5):
                blk = j + dt
                wT = conv1_w[:, :, dt].T
                if 2 <= blk <= 5:
                    w1mid = w1mid.at[128 * h + 32 * (blk - 2):
                                     128 * h + 32 * (blk - 1),
                                     col:col + 16].set(wT)
                elif blk < 2:
                    w1lo = w1lo.at[128 * h + 64 + 32 * blk:
                                   128 * h + 96 + 32 * blk,
                                   col:col + 16].set(wT)
                else:
                    w1hi = w1hi.at[128 * h + 32 * (blk - 6):
                                   128 * h + 32 * (blk - 5),
                                   col:col + 16].set(wT)
    w2mid = jnp.zeros((64, 128), f32)
    w2lo = jnp.zeros((64, 128), f32)
    w2hi = jnp.zeros((64, 128), f32)
    for h in range(2):
        for j in range(2):
            col = 64 * j + 32 * h
            for dt in range(5):
                blk = j + dt
                wT = conv2_w[:, :, dt].T
                if 2 <= blk <= 3:
                    w2mid = w2mid.at[32 * h + 16 * (blk - 2):
                                     32 * h + 16 * (blk - 1),
                                     col:col + 32].set(wT)
                elif blk < 2:
                    w2lo = w2lo.at[32 * h + 16 * blk:
                                   32 * h + 16 * (blk + 1),
                                   col:col + 32].set(wT)
                else:
                    w2hi = w2hi.at[32 * h + 16 * (blk - 4):
                                   32 * h + 16 * (blk - 3),
                                   col:col + 32].set(wT)
    b1c = jnp.tile(conv1_b, 8)[None, :]
    b2c = jnp.tile(conv2_b, 4)[None, :]

    w1blk = jnp.zeros((GL * C_IN, GL * HIDDEN), f32)
    w2blk = jnp.zeros((GL * HIDDEN, GL * HIDDEN), f32)
    fcblk = jnp.zeros((GL * HIDDEN, GL * OUT), f32)
    for g in range(GL):
        w1blk = w1blk.at[C_IN * g:C_IN * (g + 1),
                         HIDDEN * g:HIDDEN * (g + 1)].set(gcn1_w)
        w2blk = w2blk.at[HIDDEN * g:HIDDEN * (g + 1),
                         HIDDEN * g:HIDDEN * (g + 1)].set(gcn2_w)
        fcblk = fcblk.at[HIDDEN * g:HIDDEN * (g + 1),
                         OUT * g:OUT * (g + 1)].set(fc_w.T)
    g1bt = jnp.tile(gcn1_b, GL)[None, :]
    g2bt = jnp.tile(gcn2_b, GL)[None, :]
    fcbt = jnp.tile(fc_b, GL)[None, :]

    h, idx16 = pl.pallas_call(
        _conv_body,
        grid=(B // CS,),
        in_specs=[
            pl.BlockSpec((CS, U, 4 * C_IN), lambda b: (b, 0, 0)),
            pl.BlockSpec((256, 128), lambda b: (0, 0)),
            pl.BlockSpec((256, 128), lambda b: (0, 0)),
            pl.BlockSpec((256, 128), lambda b: (0, 0)),
            pl.BlockSpec((1, 128), lambda b: (0, 0)),
            pl.BlockSpec((64, 128), lambda b: (0, 0)),
            pl.BlockSpec((64, 128), lambda b: (0, 0)),
            pl.BlockSpec((64, 128), lambda b: (0, 0)),
            pl.BlockSpec((1, 128), lambda b: (0, 0)),
        ],
        out_specs=[
            pl.BlockSpec((CS, U, C_IN), lambda b: (b, 0, 0)),
            pl.BlockSpec((U, 16), lambda b: (0, 0)),
        ],
        out_shape=[
            jax.ShapeDtypeStruct((B, U, C_IN), jnp.float32),
            jax.ShapeDtypeStruct((U, 16), jnp.int32),
        ],
    )(xcat, w1lo, w1mid, w1hi, b1c, w2lo, w2mid, w2hi, b2c)

    mop = _build_m_sc(idx16)

    out = pl.pallas_call(
        _gcn_body,
        grid=(B // GB,),
        in_specs=[
            pl.BlockSpec((U, U), lambda b: (0, 0)),
            pl.BlockSpec((GB, U, C_IN), lambda b: (b, 0, 0)),
            pl.BlockSpec((GL * C_IN, GL * HIDDEN), lambda b: (0, 0)),
            pl.BlockSpec((1, GL * HIDDEN), lambda b: (0, 0)),
            pl.BlockSpec((GL * HIDDEN, GL * HIDDEN), lambda b: (0, 0)),
            pl.BlockSpec((1, GL * HIDDEN), lambda b: (0, 0)),
            pl.BlockSpec((GL * HIDDEN, GL * OUT), lambda b: (0, 0)),
            pl.BlockSpec((1, GL * OUT), lambda b: (0, 0)),
        ],
        out_specs=pl.BlockSpec((GB // GL, GL * OUT), lambda b: (b, 0)),
        out_shape=jax.ShapeDtypeStruct((B // GL, GL * OUT), jnp.float32),
    )(mop, h, w1blk, g1bt, w2blk, g2bt, fcblk, fcbt)
    return out.reshape(B, OUT)
